# Initial kernel scaffold; baseline (speedup 1.0000x reference)
import functools

import jax
import jax.numpy as jnp
from jax import lax
from jax.experimental import pallas as pl
from jax.experimental.pallas import tpu as pltpu
from jax.experimental.pallas import tpu_sc as plsc

NC = 2
NS = 16
NW = NC * NS
G = 128


def _matmul(x, wt):
  M, D = x.shape
  C = wt.shape[1]
  BM = 2048
  assert M % BM == 0

  def body(x_ref, w_ref, o_ref):
    o_ref[...] = jnp.dot(x_ref[...], w_ref[...],
                         preferred_element_type=jnp.float32)

  return pl.pallas_call(
      body,
      grid=(M // BM,),
      in_specs=[
          pl.BlockSpec((BM, D), lambda i: (i, 0)),
          pl.BlockSpec((D, C), lambda i: (0, 0)),
      ],
      out_specs=pl.BlockSpec((BM, C), lambda i: (i, 0)),
      out_shape=jax.ShapeDtypeStruct((M, C), jnp.float32),
  )(x, wt)


def _combine(p0, p1, bias, n_out):
  C = p0.shape[1]

  def body(a_ref, b_ref, bias_ref, o_ref):
    o_ref[...] = a_ref[...] + b_ref[...] + bias_ref[...]

  return pl.pallas_call(
      body,
      in_specs=[
          pl.BlockSpec((n_out, C), lambda: (0, 0)),
          pl.BlockSpec((n_out, C), lambda: (0, 0)),
          pl.BlockSpec((1, C), lambda: (0, 0)),
      ],
      out_specs=pl.BlockSpec((n_out, C), lambda: (0, 0)),
      out_shape=jax.ShapeDtypeStruct((n_out, C), jnp.float32),
  )(p0, p1, bias)


def _make_propagate(n_y, n_acc, c, n_g):
  mesh = plsc.VectorSubcoreMesh(core_axis_name="c", subcore_axis_name="s")
  rows_per_tile = n_acc // NS

  @functools.partial(
      pl.kernel,
      out_type=jax.ShapeDtypeStruct((NC, n_acc, c), jnp.float32),
      mesh=mesh,
      scratch_types=[
          pltpu.VMEM((n_g, G), jnp.int32),
          pltpu.VMEM((n_g, G), jnp.int32),
          pltpu.VMEM((G, c), jnp.float32),
          pltpu.VMEM_SHARED((n_acc, c), jnp.float32),
          pltpu.SemaphoreType.DMA,
      ],
  )
  def propagate(y_hbm, src_hbm, dst_hbm, zeros_hbm, out_hbm,
                src_v, dst_v, buf, acc, sem):
    cid = lax.axis_index("c")
    sid = lax.axis_index("s")
    wid = sid * NC + cid
    r0 = sid * rows_per_tile
    pltpu.sync_copy(zeros_hbm.at[pl.ds(r0, rows_per_tile)],
                    acc.at[pl.ds(r0, rows_per_tile)])
    pltpu.sync_copy(src_hbm.at[wid], src_v)
    pltpu.sync_copy(dst_hbm.at[wid], dst_v)
    plsc.subcore_barrier()

    def body(g, carry):
      pltpu.async_copy(y_hbm.at[src_v.at[g]], buf, sem).wait()
      pltpu.sync_copy(buf, acc.at[dst_v.at[g]], add=True)
      return carry

    lax.fori_loop(0, n_g, body, 0)
    plsc.subcore_barrier()
    pltpu.sync_copy(acc.at[pl.ds(r0, rows_per_tile)],
                    out_hbm.at[cid, pl.ds(r0, rows_per_tile)])

  return propagate


def kernel(feat, edge_index, W, b):
  N, D = feat.shape
  C = W.shape[0]
  E = edge_index.shape[1]

  n_g = -(-E // (NW * G))
  e_pad = NW * n_g * G
  n_mat = -(-N // 2048) * 2048
  n_acc = -(-(N + 1) // NS) * NS

  src = edge_index[0]
  dst = edge_index[1]
  pad = e_pad - E
  src_p = jnp.concatenate([src, jnp.full((pad,), N, jnp.int32)])
  dst_p = jnp.concatenate([dst, jnp.full((pad,), N, jnp.int32)])
  src_p = src_p.reshape(NW, n_g, G)
  dst_p = dst_p.reshape(NW, n_g, G)

  feat_p = jnp.pad(feat, ((0, n_mat - N), (0, 0)))
  y = _matmul(feat_p, W.T)
  zeros = jnp.zeros((n_acc, C), jnp.float32)

  prop1 = _make_propagate(n_mat, n_acc, C, n_g)
  p = prop1(y, src_p, dst_p, zeros)
  h1 = _combine(p[0], p[1], jnp.zeros((1, C), jnp.float32), n_acc)

  prop2 = _make_propagate(n_acc, n_acc, C, n_g)
  p2 = prop2(h1, src_p, dst_p, zeros)
  out = _combine(p2[0][:N], p2[1][:N], b.reshape(1, C), N)
  return out


# speedup vs baseline: 5.9419x; 5.9419x over previous
import functools

import jax
import jax.numpy as jnp
from jax import lax
from jax.experimental import pallas as pl
from jax.experimental.pallas import tpu as pltpu
from jax.experimental.pallas import tpu_sc as plsc

NC = 2
NS = 16
NW = NC * NS
G = 128


def _matmul(x, wt):
  M, D = x.shape
  C = wt.shape[1]
  BM = 2048
  assert M % BM == 0

  def body(x_ref, w_ref, o_ref):
    o_ref[...] = jnp.dot(x_ref[...], w_ref[...],
                         preferred_element_type=jnp.float32)

  return pl.pallas_call(
      body,
      grid=(M // BM,),
      in_specs=[
          pl.BlockSpec((BM, D), lambda i: (i, 0)),
          pl.BlockSpec((D, C), lambda i: (0, 0)),
      ],
      out_specs=pl.BlockSpec((BM, C), lambda i: (i, 0)),
      out_shape=jax.ShapeDtypeStruct((M, C), jnp.float32),
  )(x, wt)


def _combine(p0, p1, bias, n_out):
  C = p0.shape[1]

  def body(a_ref, b_ref, bias_ref, o_ref):
    o_ref[...] = a_ref[...] + b_ref[...] + bias_ref[...]

  return pl.pallas_call(
      body,
      in_specs=[
          pl.BlockSpec((n_out, C), lambda: (0, 0)),
          pl.BlockSpec((n_out, C), lambda: (0, 0)),
          pl.BlockSpec((1, C), lambda: (0, 0)),
      ],
      out_specs=pl.BlockSpec((n_out, C), lambda: (0, 0)),
      out_shape=jax.ShapeDtypeStruct((n_out, C), jnp.float32),
  )(p0, p1, bias)


def _make_propagate(n_y, n_acc, c, n_g):
  mesh = plsc.VectorSubcoreMesh(core_axis_name="c", subcore_axis_name="s")
  rows_per_tile = n_acc // NS

  @functools.partial(
      pl.kernel,
      out_type=jax.ShapeDtypeStruct((NC, n_acc, c), jnp.float32),
      mesh=mesh,
      scratch_types=[
          pltpu.VMEM((n_g, G), jnp.int32),
          pltpu.VMEM((n_g, G), jnp.int32),
          pltpu.VMEM((G, c), jnp.float32),
          pltpu.VMEM_SHARED((n_acc, c), jnp.float32),
          pltpu.SemaphoreType.DMA,
      ],
      compiler_params=pltpu.CompilerParams(use_tc_tiling_on_sc=False),
  )
  def propagate(y_hbm, src_hbm, dst_hbm, zeros_hbm, out_hbm,
                src_v, dst_v, buf, acc, sem):
    cid = lax.axis_index("c")
    sid = lax.axis_index("s")
    wid = sid * NC + cid
    r0 = sid * rows_per_tile
    pltpu.sync_copy(zeros_hbm.at[pl.ds(r0, rows_per_tile)],
                    acc.at[pl.ds(r0, rows_per_tile)])
    pltpu.sync_copy(src_hbm.at[wid], src_v)
    pltpu.sync_copy(dst_hbm.at[wid], dst_v)
    plsc.subcore_barrier()

    def body(g, carry):
      pltpu.async_copy(y_hbm.at[src_v.at[g]], buf, sem).wait()
      pltpu.sync_copy(buf, acc.at[dst_v.at[g]], add=True)
      return carry

    lax.fori_loop(0, n_g, body, 0)
    plsc.subcore_barrier()
    pltpu.sync_copy(acc.at[pl.ds(r0, rows_per_tile)],
                    out_hbm.at[cid, pl.ds(r0, rows_per_tile)])

  return propagate


def kernel(feat, edge_index, W, b):
  N, D = feat.shape
  C = W.shape[0]
  E = edge_index.shape[1]

  n_g = -(-E // (NW * G))
  e_pad = NW * n_g * G
  n_mat = -(-N // 2048) * 2048
  n_acc = -(-(N + 1) // (NS * 8)) * NS * 8

  src = edge_index[0]
  dst = edge_index[1]
  pad = e_pad - E
  src_p = jnp.concatenate([src, jnp.full((pad,), N, jnp.int32)])
  dst_p = jnp.concatenate([dst, jnp.full((pad,), N, jnp.int32)])
  src_p = src_p.reshape(NW, n_g, G)
  dst_p = dst_p.reshape(NW, n_g, G)

  feat_p = jnp.pad(feat, ((0, n_mat - N), (0, 0)))
  y = _matmul(feat_p, W.T)
  zeros = jnp.zeros((n_acc, C), jnp.float32)

  prop1 = _make_propagate(n_mat, n_acc, C, n_g)
  p = prop1(y, src_p, dst_p, zeros)
  h1 = _combine(p[0], p[1], jnp.zeros((1, C), jnp.float32), n_acc)

  prop2 = _make_propagate(n_acc, n_acc, C, n_g)
  p2 = prop2(h1, src_p, dst_p, zeros)
  out = _combine(p2[0][:N], p2[1][:N], b.reshape(1, C), N)
  return out


# speedup vs baseline: 6.7871x; 1.1422x over previous
import functools

import jax
import jax.numpy as jnp
from jax import lax
from jax.experimental import pallas as pl
from jax.experimental.pallas import tpu as pltpu
from jax.experimental.pallas import tpu_sc as plsc

NC = 2
NS = 16
NW = NC * NS
G = 128


def _matmul(x, wt):
  M, D = x.shape
  C = wt.shape[1]
  BM = 2048
  assert M % BM == 0

  def body(x_ref, w_ref, o_ref):
    o_ref[...] = jnp.dot(x_ref[...], w_ref[...],
                         preferred_element_type=jnp.float32)

  return pl.pallas_call(
      body,
      grid=(M // BM,),
      in_specs=[
          pl.BlockSpec((BM, D), lambda i: (i, 0)),
          pl.BlockSpec((D, C), lambda i: (0, 0)),
      ],
      out_specs=pl.BlockSpec((BM, C), lambda i: (i, 0)),
      out_shape=jax.ShapeDtypeStruct((M, C), jnp.float32),
  )(x, wt)


def _combine(p0, p1, bias, n_out):
  C = p0.shape[1]

  def body(a_ref, b_ref, bias_ref, o_ref):
    o_ref[...] = a_ref[...] + b_ref[...] + bias_ref[...]

  return pl.pallas_call(
      body,
      in_specs=[
          pl.BlockSpec((n_out, C), lambda: (0, 0)),
          pl.BlockSpec((n_out, C), lambda: (0, 0)),
          pl.BlockSpec((1, C), lambda: (0, 0)),
      ],
      out_specs=pl.BlockSpec((n_out, C), lambda: (0, 0)),
      out_shape=jax.ShapeDtypeStruct((n_out, C), jnp.float32),
  )(p0, p1, bias)


def _make_propagate(n_y, n_acc, c, n_g):
  mesh = plsc.VectorSubcoreMesh(core_axis_name="c", subcore_axis_name="s")
  rows_per_tile = n_acc // NS
  NB = 4
  assert n_g % NB == 0

  @functools.partial(
      pl.kernel,
      out_type=jax.ShapeDtypeStruct((NC, n_acc, c), jnp.float32),
      mesh=mesh,
      scratch_types=[
          pltpu.VMEM((n_g, G), jnp.int32),
          pltpu.VMEM((n_g, G), jnp.int32),
          [pltpu.VMEM((G, c), jnp.float32)] * NB,
          pltpu.VMEM_SHARED((n_acc, c), jnp.float32),
          [pltpu.SemaphoreType.DMA] * NB,
          [pltpu.SemaphoreType.DMA] * NB,
      ],
      compiler_params=pltpu.CompilerParams(use_tc_tiling_on_sc=False),
  )
  def propagate(y_hbm, src_hbm, dst_hbm, zeros_hbm, out_hbm,
                src_v, dst_v, bufs, acc, gsems, ssems):
    cid = lax.axis_index("c")
    sid = lax.axis_index("s")
    wid = sid * NC + cid
    r0 = sid * rows_per_tile
    pltpu.sync_copy(zeros_hbm.at[pl.ds(r0, rows_per_tile)],
                    acc.at[pl.ds(r0, rows_per_tile)])
    pltpu.sync_copy(src_hbm.at[wid], src_v)
    pltpu.sync_copy(dst_hbm.at[wid], dst_v)
    plsc.subcore_barrier()

    for ph in range(NB):
      pltpu.async_copy(y_hbm.at[src_v.at[ph]], bufs[ph], gsems[ph])

    def body(i, carry):
      base = i * NB
      for ph in range(NB):
        g = base + ph
        pltpu.make_async_copy(y_hbm.at[src_v.at[g]], bufs[ph],
                              gsems[ph]).wait()
        pltpu.async_copy(bufs[ph], acc.at[dst_v.at[g]], ssems[ph],
                         add=True)
      for ph in range(NB):
        g = base + ph
        pltpu.make_async_copy(bufs[ph], acc.at[dst_v.at[g]],
                              ssems[ph]).wait()

        @pl.when(g + NB < n_g)
        def _():
          pltpu.async_copy(y_hbm.at[src_v.at[g + NB]], bufs[ph],
                           gsems[ph])

      return carry

    lax.fori_loop(0, n_g // NB, body, 0)
    plsc.subcore_barrier()
    pltpu.sync_copy(acc.at[pl.ds(r0, rows_per_tile)],
                    out_hbm.at[cid, pl.ds(r0, rows_per_tile)])

  return propagate


def kernel(feat, edge_index, W, b):
  N, D = feat.shape
  C = W.shape[0]
  E = edge_index.shape[1]

  n_g = -(-E // (NW * G))
  n_g = -(-n_g // 4) * 4
  e_pad = NW * n_g * G
  n_mat = -(-N // 2048) * 2048
  n_acc = -(-(N + 1) // (NS * 8)) * NS * 8

  src = edge_index[0]
  dst = edge_index[1]
  pad = e_pad - E
  src_p = jnp.concatenate([src, jnp.full((pad,), N, jnp.int32)])
  dst_p = jnp.concatenate([dst, jnp.full((pad,), N, jnp.int32)])
  src_p = src_p.reshape(NW, n_g, G)
  dst_p = dst_p.reshape(NW, n_g, G)

  feat_p = jnp.pad(feat, ((0, n_mat - N), (0, 0)))
  y = _matmul(feat_p, W.T)
  zeros = jnp.zeros((n_acc, C), jnp.float32)

  prop1 = _make_propagate(n_mat, n_acc, C, n_g)
  p = prop1(y, src_p, dst_p, zeros)
  h1 = _combine(p[0], p[1], jnp.zeros((1, C), jnp.float32), n_acc)

  prop2 = _make_propagate(n_acc, n_acc, C, n_g)
  p2 = prop2(h1, src_p, dst_p, zeros)
  out = _combine(p2[0][:N], p2[1][:N], b.reshape(1, C), N)
  return out


# speedup vs baseline: 10.7067x; 1.5775x over previous
import functools

import jax
import jax.numpy as jnp
from jax import lax
from jax.experimental import pallas as pl
from jax.experimental.pallas import tpu as pltpu
from jax.experimental.pallas import tpu_sc as plsc

NC = 2
NS = 16
NW = NC * NS
G = 128


def _matmul(x, wt):
  M, D = x.shape
  C = wt.shape[1]
  BM = 2048
  assert M % BM == 0

  def body(x_ref, w_ref, o_ref):
    o_ref[...] = jnp.dot(x_ref[...], w_ref[...],
                         preferred_element_type=jnp.float32)

  return pl.pallas_call(
      body,
      grid=(M // BM,),
      in_specs=[
          pl.BlockSpec((BM, D), lambda i: (i, 0)),
          pl.BlockSpec((D, C), lambda i: (0, 0)),
      ],
      out_specs=pl.BlockSpec((BM, C), lambda i: (i, 0)),
      out_shape=jax.ShapeDtypeStruct((M, C), jnp.float32),
  )(x, wt)


def _combine(p0, p1, bias, n_out):
  C = p0.shape[1]

  def body(a_ref, b_ref, bias_ref, o_ref):
    o_ref[...] = a_ref[...] + b_ref[...] + bias_ref[...]

  return pl.pallas_call(
      body,
      in_specs=[
          pl.BlockSpec((n_out, C), lambda: (0, 0)),
          pl.BlockSpec((n_out, C), lambda: (0, 0)),
          pl.BlockSpec((1, C), lambda: (0, 0)),
      ],
      out_specs=pl.BlockSpec((n_out, C), lambda: (0, 0)),
      out_shape=jax.ShapeDtypeStruct((n_out, C), jnp.float32),
  )(p0, p1, bias)


def _make_propagate(n_y, n_acc, c, n_g):
  mesh = plsc.VectorSubcoreMesh(core_axis_name="c", subcore_axis_name="s")
  rows_per_tile = n_acc // NS
  y_rows_per_tile = n_y // NS
  NB = 4
  assert n_g % NB == 0
  assert n_y % NS == 0 and y_rows_per_tile % 8 == 0

  @functools.partial(
      pl.kernel,
      out_type=jax.ShapeDtypeStruct((NC, n_acc, c), jnp.float32),
      mesh=mesh,
      scratch_types=[
          pltpu.VMEM((n_g, G), jnp.int32),
          pltpu.VMEM((n_g, G), jnp.int32),
          [pltpu.VMEM((G, c), jnp.float32)] * NB,
          pltpu.VMEM_SHARED((n_y, c), jnp.float32),
          pltpu.VMEM_SHARED((n_acc, c), jnp.float32),
          [pltpu.SemaphoreType.DMA] * NB,
          [pltpu.SemaphoreType.DMA] * NB,
      ],
      compiler_params=pltpu.CompilerParams(use_tc_tiling_on_sc=False),
  )
  def propagate(y_hbm, src_hbm, dst_hbm, zeros_hbm, out_hbm,
                src_v, dst_v, bufs, y_sp, acc, gsems, ssems):
    cid = lax.axis_index("c")
    sid = lax.axis_index("s")
    wid = sid * NC + cid
    r0 = sid * rows_per_tile
    yr0 = sid * y_rows_per_tile
    pltpu.sync_copy(y_hbm.at[pl.ds(yr0, y_rows_per_tile)],
                    y_sp.at[pl.ds(yr0, y_rows_per_tile)])
    pltpu.sync_copy(zeros_hbm.at[pl.ds(r0, rows_per_tile)],
                    acc.at[pl.ds(r0, rows_per_tile)])
    pltpu.sync_copy(src_hbm.at[wid], src_v)
    pltpu.sync_copy(dst_hbm.at[wid], dst_v)
    plsc.subcore_barrier()

    for ph in range(NB):
      pltpu.async_copy(y_sp.at[src_v.at[ph]], bufs[ph], gsems[ph])

    def body(i, carry):
      base = i * NB
      for ph in range(NB):
        g = base + ph
        pltpu.make_async_copy(y_sp.at[src_v.at[g]], bufs[ph],
                              gsems[ph]).wait()
        pltpu.async_copy(bufs[ph], acc.at[dst_v.at[g]], ssems[ph],
                         add=True)
      for ph in range(NB):
        g = base + ph
        pltpu.make_async_copy(bufs[ph], acc.at[dst_v.at[g]],
                              ssems[ph]).wait()

        @pl.when(g + NB < n_g)
        def _():
          pltpu.async_copy(y_sp.at[src_v.at[g + NB]], bufs[ph],
                           gsems[ph])

      return carry

    lax.fori_loop(0, n_g // NB, body, 0)
    plsc.subcore_barrier()
    pltpu.sync_copy(acc.at[pl.ds(r0, rows_per_tile)],
                    out_hbm.at[cid, pl.ds(r0, rows_per_tile)])

  return propagate


def kernel(feat, edge_index, W, b):
  N, D = feat.shape
  C = W.shape[0]
  E = edge_index.shape[1]

  n_g = -(-E // (NW * G))
  n_g = -(-n_g // 4) * 4
  e_pad = NW * n_g * G
  n_mat = -(-N // 2048) * 2048
  n_acc = -(-(N + 1) // (NS * 8)) * NS * 8

  src = edge_index[0]
  dst = edge_index[1]
  pad = e_pad - E
  src_p = jnp.concatenate([src, jnp.full((pad,), N, jnp.int32)])
  dst_p = jnp.concatenate([dst, jnp.full((pad,), N, jnp.int32)])
  src_p = src_p.reshape(NW, n_g, G)
  dst_p = dst_p.reshape(NW, n_g, G)

  feat_p = jnp.pad(feat, ((0, n_mat - N), (0, 0)))
  y = _matmul(feat_p, W.T)
  zeros = jnp.zeros((n_acc, C), jnp.float32)

  prop1 = _make_propagate(n_mat, n_acc, C, n_g)
  p = prop1(y, src_p, dst_p, zeros)
  h1 = _combine(p[0], p[1], jnp.zeros((1, C), jnp.float32), n_acc)

  prop2 = _make_propagate(n_acc, n_acc, C, n_g)
  p2 = prop2(h1, src_p, dst_p, zeros)
  out = _combine(p2[0][:N], p2[1][:N], b.reshape(1, C), N)
  return out


# speedup vs baseline: 13.1104x; 1.2245x over previous
import functools

import jax
import jax.numpy as jnp
from jax import lax
from jax.experimental import pallas as pl
from jax.experimental.pallas import tpu as pltpu
from jax.experimental.pallas import tpu_sc as plsc

NC = 2
NS = 16
NW = NC * NS
G = 128
NB = 4


def _matmul(x, wt, m_out):
  _, D = x.shape
  C = wt.shape[1]
  BM = m_out // 8
  assert BM % 8 == 0

  def body(x_ref, w_ref, o_ref):
    o_ref[...] = jnp.dot(x_ref[...], w_ref[...],
                         preferred_element_type=jnp.float32)

  return pl.pallas_call(
      body,
      grid=(m_out // BM,),
      in_specs=[
          pl.BlockSpec((BM, D), lambda i: (i, 0)),
          pl.BlockSpec((D, C), lambda i: (0, 0)),
      ],
      out_specs=pl.BlockSpec((BM, C), lambda i: (i, 0)),
      out_shape=jax.ShapeDtypeStruct((m_out, C), jnp.float32),
  )(x, wt)


def _combine(p, bias, n_out):
  n_acc, C = p.shape[1:]

  def body(p_ref, bias_ref, o_ref):
    o_ref[...] = p_ref[0] + p_ref[1] + bias_ref[...]

  return pl.pallas_call(
      body,
      grid=(1,),
      in_specs=[
          pl.BlockSpec((2, n_out, C), lambda i: (0, 0, 0)),
          pl.BlockSpec((1, C), lambda i: (0, 0)),
      ],
      out_specs=pl.BlockSpec((n_out, C), lambda i: (0, 0)),
      out_shape=jax.ShapeDtypeStruct((n_out, C), jnp.float32),
  )(p, bias)


def _make_propagate(n_acc, c, n_g, two_partials):
  mesh = plsc.VectorSubcoreMesh(core_axis_name="c", subcore_axis_name="s")
  rows_per_tile = n_acc // NS
  n_row_g = n_acc // G
  kmax = -(-n_row_g // NS)
  assert n_acc % (NS * 8) == 0 and n_acc % G == 0
  assert n_g % NB == 0

  @functools.partial(
      pl.kernel,
      out_type=jax.ShapeDtypeStruct((NC, n_acc, c), jnp.float32),
      mesh=mesh,
      scratch_types=[
          pltpu.VMEM((n_g, G), jnp.int32),
          pltpu.VMEM((n_g, G), jnp.int32),
          pltpu.VMEM((G,), jnp.int32),
          [pltpu.VMEM((G, c), jnp.float32)] * NB,
          pltpu.VMEM_SHARED((n_acc, c), jnp.float32),
          pltpu.VMEM_SHARED((n_acc, c), jnp.float32),
          [pltpu.SemaphoreType.DMA] * NB,
          [pltpu.SemaphoreType.DMA] * NB,
      ],
      compiler_params=pltpu.CompilerParams(use_tc_tiling_on_sc=False),
  )
  def propagate(y_hbm, ep_hbm, zeros_hbm, iota_hbm, out_hbm,
                src_v, dst_v, idx_v, bufs, y_sp, acc, gsems, ssems):
    cid = lax.axis_index("c")
    sid = lax.axis_index("s")
    wid = sid * NC + cid
    r0 = sid * rows_per_tile
    slab = pl.ds(r0, rows_per_tile)
    if two_partials:
      pltpu.sync_copy(y_hbm.at[0, slab], y_sp.at[slab])
    else:
      pltpu.sync_copy(y_hbm.at[slab], y_sp.at[slab])
    pltpu.sync_copy(zeros_hbm.at[slab], acc.at[slab])
    pltpu.sync_copy(ep_hbm.at[0, wid], src_v)
    pltpu.sync_copy(ep_hbm.at[1, wid], dst_v)
    plsc.subcore_barrier()

    if two_partials:
      for k in range(kmax):
        gr = sid * kmax + k

        @pl.when(gr < n_row_g)
        def _():
          pltpu.sync_copy(iota_hbm.at[gr], idx_v)
          pltpu.sync_copy(y_hbm.at[1, pl.ds(gr * G, G)], bufs[0])
          pltpu.sync_copy(bufs[0], y_sp.at[idx_v], add=True)

      plsc.subcore_barrier()

    for ph in range(NB):
      pltpu.async_copy(y_sp.at[src_v.at[ph]], bufs[ph], gsems[ph])

    def body(i, carry):
      base = i * NB
      for ph in range(NB):
        g = base + ph
        pltpu.make_async_copy(y_sp.at[src_v.at[g]], bufs[ph],
                              gsems[ph]).wait()
        pltpu.async_copy(bufs[ph], acc.at[dst_v.at[g]], ssems[ph],
                         add=True)
      for ph in range(NB):
        g = base + ph
        pltpu.make_async_copy(bufs[ph], acc.at[dst_v.at[g]],
                              ssems[ph]).wait()

        @pl.when(g + NB < n_g)
        def _():
          pltpu.async_copy(y_sp.at[src_v.at[g + NB]], bufs[ph],
                           gsems[ph])

      return carry

    lax.fori_loop(0, n_g // NB, body, 0)
    plsc.subcore_barrier()
    pltpu.sync_copy(acc.at[slab], out_hbm.at[cid, slab])

  return propagate


def kernel(feat, edge_index, W, b):
  N, D = feat.shape
  C = W.shape[0]
  E = edge_index.shape[1]

  n_g = -(-E // (NW * G))
  n_g = -(-n_g // NB) * NB
  e_pad = NW * n_g * G
  n_acc = -(-(N + 1) // (NS * 8)) * NS * 8
  n_acc = -(-n_acc // G) * G

  ep = jnp.pad(edge_index, ((0, 0), (0, e_pad - E)), constant_values=N)
  ep = ep.reshape(2, NW, n_g, G)

  y = _matmul(feat, W.T, n_acc)
  zeros = jnp.zeros((n_acc, C), jnp.float32)
  iota = jnp.arange(n_acc, dtype=jnp.int32).reshape(n_acc // G, G)

  prop1 = _make_propagate(n_acc, C, n_g, two_partials=False)
  p = prop1(y, ep, zeros, iota)
  prop2 = _make_propagate(n_acc, C, n_g, two_partials=True)
  p2 = prop2(p, ep, zeros, iota)
  return _combine(p2, b.reshape(1, C), N)


# speedup vs baseline: 13.1764x; 1.0050x over previous
import functools

import jax
import jax.numpy as jnp
from jax import lax
from jax.experimental import pallas as pl
from jax.experimental.pallas import tpu as pltpu
from jax.experimental.pallas import tpu_sc as plsc

NC = 2
NS = 16
NW = NC * NS
G = 128
NB = 4
L = 16


def _matmul(x, wt, m_out):
  _, D = x.shape
  C = wt.shape[1]
  BM = m_out // 16
  assert BM % 8 == 0

  def body(x_ref, w_ref, o_ref):
    o_ref[...] = jnp.dot(x_ref[...], w_ref[...],
                         preferred_element_type=jnp.float32)

  return pl.pallas_call(
      body,
      grid=(m_out // BM,),
      in_specs=[
          pl.BlockSpec((BM, D), lambda i: (i, 0)),
          pl.BlockSpec((D, C), lambda i: (0, 0)),
      ],
      out_specs=pl.BlockSpec((BM, C), lambda i: (i, 0)),
      out_shape=jax.ShapeDtypeStruct((m_out, C), jnp.float32),
  )(x, wt)


def _make_propagate(n_acc, c, n_g, two_partials):
  mesh = plsc.VectorSubcoreMesh(core_axis_name="c", subcore_axis_name="s")
  rows_per_tile = n_acc // NS
  n_full = rows_per_tile // G
  rem = rows_per_tile - n_full * G
  assert n_acc % (NS * 8) == 0 and rem % 8 == 0
  assert n_g % NB == 0 and n_full <= NB

  @functools.partial(
      pl.kernel,
      out_type=jax.ShapeDtypeStruct((NC, n_acc, c), jnp.float32),
      mesh=mesh,
      scratch_types=[
          pltpu.VMEM((n_g, G), jnp.int32),
          pltpu.VMEM((n_g, G), jnp.int32),
          pltpu.VMEM((G,), jnp.int32),
          pltpu.VMEM((max(rem, 8),), jnp.int32),
          [pltpu.VMEM((G, c), jnp.float32)] * NB,
          pltpu.VMEM_SHARED((n_acc, c), jnp.float32),
          pltpu.VMEM_SHARED((n_acc, c), jnp.float32),
          [pltpu.SemaphoreType.DMA] * NB,
          [pltpu.SemaphoreType.DMA] * NB,
      ],
      compiler_params=pltpu.CompilerParams(use_tc_tiling_on_sc=False),
  )
  def propagate(y_hbm, ep_hbm, zeros_hbm, iota_hbm, out_hbm,
                src_v, dst_v, idx_v, idx_r, bufs, y_sp, acc,
                gsems, ssems):
    cid = lax.axis_index("c")
    sid = lax.axis_index("s")
    wid = sid * NC + cid
    r0 = sid * rows_per_tile
    slab = pl.ds(r0, rows_per_tile)
    if two_partials:
      pltpu.sync_copy(y_hbm.at[0, slab], y_sp.at[slab])
      for k in range(n_full):
        pltpu.async_copy(y_hbm.at[1, pl.ds(r0 + k * G, G)], bufs[k],
                         gsems[k])
      for k in range(n_full):
        pltpu.sync_copy(iota_hbm.at[pl.ds(r0 + k * G, G)], idx_v)
        pltpu.make_async_copy(y_hbm.at[1, pl.ds(r0 + k * G, G)], bufs[k],
                              gsems[k]).wait()
        pltpu.sync_copy(bufs[k], y_sp.at[idx_v], add=True)
      if rem:
        pltpu.sync_copy(iota_hbm.at[pl.ds(r0 + n_full * G, rem)], idx_r)
        pltpu.sync_copy(y_hbm.at[1, pl.ds(r0 + n_full * G, rem)],
                        bufs[0].at[pl.ds(0, rem)])
        pltpu.sync_copy(bufs[0].at[pl.ds(0, rem)], y_sp.at[idx_r],
                        add=True)
    else:
      pltpu.sync_copy(y_hbm.at[slab], y_sp.at[slab])
    pltpu.sync_copy(zeros_hbm.at[slab], acc.at[slab])
    pltpu.sync_copy(ep_hbm.at[0, wid], src_v)
    pltpu.sync_copy(ep_hbm.at[1, wid], dst_v)
    plsc.subcore_barrier()

    for ph in range(NB):
      pltpu.async_copy(y_sp.at[src_v.at[ph]], bufs[ph], gsems[ph])

    def body(i, carry):
      base = i * NB
      for ph in range(NB):
        g = base + ph
        pltpu.make_async_copy(y_sp.at[src_v.at[g]], bufs[ph],
                              gsems[ph]).wait()
        pltpu.async_copy(bufs[ph], acc.at[dst_v.at[g]], ssems[ph],
                         add=True)
      for ph in range(NB):
        g = base + ph
        pltpu.make_async_copy(bufs[ph], acc.at[dst_v.at[g]],
                              ssems[ph]).wait()

        @pl.when(g + NB < n_g)
        def _():
          pltpu.async_copy(y_sp.at[src_v.at[g + NB]], bufs[ph],
                           gsems[ph])

      return carry

    lax.fori_loop(0, n_g // NB, body, 0)
    plsc.subcore_barrier()
    pltpu.sync_copy(acc.at[slab], out_hbm.at[cid, slab])

  return propagate


def _make_final_combine(n_acc, c):
  mesh = plsc.VectorSubcoreMesh(core_axis_name="c", subcore_axis_name="s")
  rows_per_w = n_acc // NW
  assert n_acc % NW == 0 and c % L == 0

  @functools.partial(
      pl.kernel,
      out_type=jax.ShapeDtypeStruct((n_acc, c), jnp.float32),
      mesh=mesh,
      scratch_types=[
          pltpu.VMEM((rows_per_w, c), jnp.float32),
          pltpu.VMEM((rows_per_w, c), jnp.float32),
          pltpu.VMEM((c,), jnp.float32),
          pltpu.SemaphoreType.DMA,
          pltpu.SemaphoreType.DMA,
      ],
      compiler_params=pltpu.CompilerParams(use_tc_tiling_on_sc=False),
  )
  def combine(p_hbm, b_hbm, out_hbm, a_buf, b_buf, bias_v, sem_a, sem_b):
    cid = lax.axis_index("c")
    sid = lax.axis_index("s")
    wid = sid * NC + cid
    r0 = wid * rows_per_w
    rows = pl.ds(r0, rows_per_w)
    pltpu.async_copy(p_hbm.at[0, rows], a_buf, sem_a)
    pltpu.async_copy(p_hbm.at[1, rows], b_buf, sem_b)
    pltpu.sync_copy(b_hbm, bias_v)
    bias = [bias_v[pl.ds(l * L, L)] for l in range(c // L)]
    pltpu.make_async_copy(p_hbm.at[0, rows], a_buf, sem_a).wait()
    pltpu.make_async_copy(p_hbm.at[1, rows], b_buf, sem_b).wait()

    def body(r, carry):
      for l in range(c // L):
        cols = pl.ds(l * L, L)
        a_buf[r, cols] = a_buf[r, cols] + b_buf[r, cols] + bias[l]
      return carry

    lax.fori_loop(0, rows_per_w, body, 0)
    pltpu.sync_copy(a_buf, out_hbm.at[rows])

  return combine


def kernel(feat, edge_index, W, b):
  N, D = feat.shape
  C = W.shape[0]
  E = edge_index.shape[1]

  n_g = -(-E // (NW * G))
  n_g = -(-n_g // NB) * NB
  e_pad = NW * n_g * G
  n_acc = -(-(N + 1) // (NS * 8)) * NS * 8
  n_acc = -(-n_acc // NW) * NW

  ep = jnp.pad(edge_index, ((0, 0), (0, e_pad - E)), constant_values=N)
  ep = ep.reshape(2, NW, n_g, G)

  y = _matmul(feat, W.T, n_acc)
  zeros = jnp.zeros((n_acc, C), jnp.float32)
  iota = jnp.arange(n_acc, dtype=jnp.int32)

  prop1 = _make_propagate(n_acc, C, n_g, two_partials=False)
  p = prop1(y, ep, zeros, iota)
  prop2 = _make_propagate(n_acc, C, n_g, two_partials=True)
  p2 = prop2(p, ep, zeros, iota)
  out = _make_final_combine(n_acc, C)(p2, b)
  return out[:N]


# speedup vs baseline: 13.5500x; 1.0284x over previous
import functools

import numpy as np

import jax
import jax.numpy as jnp
from jax import lax
from jax.experimental import pallas as pl
from jax.experimental.pallas import tpu as pltpu
from jax.experimental.pallas import tpu_sc as plsc

NC = 2
NS = 16
NW = NC * NS
G = 128
NB = 4
L = 16


def _matmul(x, wt, m_out):
  _, D = x.shape
  C = wt.shape[1]
  BM = m_out // 8
  assert BM % 8 == 0

  def body(x_ref, w_ref, o_ref):
    o_ref[...] = jnp.dot(x_ref[...], w_ref[...],
                         preferred_element_type=jnp.float32)

  return pl.pallas_call(
      body,
      grid=(m_out // BM,),
      in_specs=[
          pl.BlockSpec((BM, D), lambda i: (i, 0)),
          pl.BlockSpec((D, C), lambda i: (0, 0)),
      ],
      out_specs=pl.BlockSpec((BM, C), lambda i: (i, 0)),
      out_shape=jax.ShapeDtypeStruct((m_out, C), jnp.float32),
  )(x, wt)


def _make_propagate(n_acc, c, n_g, two_partials):
  mesh = plsc.VectorSubcoreMesh(core_axis_name="c", subcore_axis_name="s")
  rows_per_tile = n_acc // NS
  n_full = rows_per_tile // G
  rem = rows_per_tile - n_full * G
  assert n_acc % (NS * 8) == 0 and rem % 8 == 0
  assert n_g % NB == 0 and n_full <= NB

  @functools.partial(
      pl.kernel,
      out_type=jax.ShapeDtypeStruct((NC, n_acc, c), jnp.float32),
      mesh=mesh,
      scratch_types=[
          pltpu.VMEM((n_g, G), jnp.int32),
          pltpu.VMEM((n_g, G), jnp.int32),
          pltpu.VMEM((n_full, G), jnp.int32),
          pltpu.VMEM((max(rem, 8),), jnp.int32),
          [pltpu.VMEM((G, c), jnp.float32)] * NB,
          pltpu.VMEM_SHARED((n_acc, c), jnp.float32),
          pltpu.VMEM_SHARED((n_acc, c), jnp.float32),
          [pltpu.SemaphoreType.DMA] * NB,
          [pltpu.SemaphoreType.DMA] * NB,
      ],
      compiler_params=pltpu.CompilerParams(use_tc_tiling_on_sc=False),
  )
  def propagate(y_hbm, ep_hbm, zeros_hbm, iota2_hbm, iotar_hbm, out_hbm,
                src_v, dst_v, idx_v, idx_r, bufs, y_sp, acc,
                gsems, ssems):
    cid = lax.axis_index("c")
    sid = lax.axis_index("s")
    wid = sid * NC + cid
    r0 = sid * rows_per_tile
    slab = pl.ds(r0, rows_per_tile)
    if two_partials:
      for k in range(n_full):
        pltpu.async_copy(y_hbm.at[1, pl.ds(r0 + k * G, G)], bufs[k],
                         gsems[k])
      pltpu.async_copy(iota2_hbm.at[sid], idx_v, ssems[0])
      pltpu.async_copy(iotar_hbm.at[sid], idx_r, ssems[1])
      pltpu.sync_copy(y_hbm.at[0, slab], y_sp.at[slab])
      pltpu.make_async_copy(iota2_hbm.at[sid], idx_v, ssems[0]).wait()
      pltpu.make_async_copy(iotar_hbm.at[sid], idx_r, ssems[1]).wait()
      for k in range(n_full):
        pltpu.make_async_copy(y_hbm.at[1, pl.ds(r0 + k * G, G)],
                              bufs[k], gsems[k]).wait()
        pltpu.sync_copy(bufs[k], y_sp.at[idx_v.at[k]], add=True)
      if rem:
        pltpu.sync_copy(y_hbm.at[1, pl.ds(r0 + n_full * G, rem)],
                        bufs[0].at[pl.ds(0, rem)])
        pltpu.sync_copy(bufs[0].at[pl.ds(0, rem)], y_sp.at[idx_r],
                        add=True)
    else:
      pltpu.sync_copy(y_hbm.at[slab], y_sp.at[slab])
    pltpu.sync_copy(zeros_hbm.at[slab], acc.at[slab])
    pltpu.sync_copy(ep_hbm.at[0, wid], src_v)
    pltpu.sync_copy(ep_hbm.at[1, wid], dst_v)
    plsc.subcore_barrier()

    for ph in range(NB):
      pltpu.async_copy(y_sp.at[src_v.at[ph]], bufs[ph], gsems[ph])

    def body(i, carry):
      base = i * NB
      for ph in range(NB):
        g = base + ph
        pltpu.make_async_copy(y_sp.at[src_v.at[g]], bufs[ph],
                              gsems[ph]).wait()
        pltpu.async_copy(bufs[ph], acc.at[dst_v.at[g]], ssems[ph],
                         add=True)
      for ph in range(NB):
        g = base + ph
        pltpu.make_async_copy(bufs[ph], acc.at[dst_v.at[g]],
                              ssems[ph]).wait()

        @pl.when(g + NB < n_g)
        def _():
          pltpu.async_copy(y_sp.at[src_v.at[g + NB]], bufs[ph],
                           gsems[ph])

      return carry

    lax.fori_loop(0, n_g // NB, body, 0)
    plsc.subcore_barrier()
    pltpu.sync_copy(acc.at[slab], out_hbm.at[cid, slab])

  return propagate


def _make_final_combine(n_acc, c):
  mesh = plsc.VectorSubcoreMesh(core_axis_name="c", subcore_axis_name="s")
  rows_per_w = n_acc // NW
  assert n_acc % NW == 0 and c % L == 0

  @functools.partial(
      pl.kernel,
      out_type=jax.ShapeDtypeStruct((n_acc, c), jnp.float32),
      mesh=mesh,
      scratch_types=[
          pltpu.VMEM((rows_per_w, c), jnp.float32),
          pltpu.VMEM((rows_per_w, c), jnp.float32),
          pltpu.VMEM((c,), jnp.float32),
          pltpu.SemaphoreType.DMA,
          pltpu.SemaphoreType.DMA,
      ],
      compiler_params=pltpu.CompilerParams(use_tc_tiling_on_sc=False),
  )
  def combine(p_hbm, b_hbm, out_hbm, a_buf, b_buf, bias_v, sem_a, sem_b):
    cid = lax.axis_index("c")
    sid = lax.axis_index("s")
    wid = sid * NC + cid
    r0 = wid * rows_per_w
    rows = pl.ds(r0, rows_per_w)
    pltpu.async_copy(p_hbm.at[0, rows], a_buf, sem_a)
    pltpu.async_copy(p_hbm.at[1, rows], b_buf, sem_b)
    pltpu.sync_copy(b_hbm, bias_v)
    bias = [bias_v[pl.ds(l * L, L)] for l in range(c // L)]
    pltpu.make_async_copy(p_hbm.at[0, rows], a_buf, sem_a).wait()
    pltpu.make_async_copy(p_hbm.at[1, rows], b_buf, sem_b).wait()

    def body(r, carry):
      for l in range(c // L):
        cols = pl.ds(l * L, L)
        a_buf[r, cols] = a_buf[r, cols] + b_buf[r, cols] + bias[l]
      return carry

    lax.fori_loop(0, rows_per_w, body, 0)
    pltpu.sync_copy(a_buf, out_hbm.at[rows])

  return combine


def kernel(feat, edge_index, W, b):
  N, D = feat.shape
  C = W.shape[0]
  E = edge_index.shape[1]

  n_g = -(-E // (NW * G))
  n_g = -(-n_g // NB) * NB
  e_pad = NW * n_g * G
  n_acc = -(-(N + 1) // (NS * 8)) * NS * 8
  n_acc = -(-n_acc // NW) * NW

  ep = jnp.pad(edge_index, ((0, 0), (0, e_pad - E)), constant_values=N)
  ep = ep.reshape(2, NW, n_g, G)

  y = _matmul(feat, W.T, n_acc)
  zeros = jnp.asarray(np.zeros((n_acc, C), np.float32))
  rows_per_tile = n_acc // NS
  n_full = rows_per_tile // G
  rem = rows_per_tile - n_full * G
  base = np.arange(NS, dtype=np.int32)[:, None] * rows_per_tile
  iota2 = base[:, :, None] + np.arange(n_full * G, dtype=np.int32
                                       ).reshape(1, n_full, G)
  iotar = base + n_full * G + np.arange(max(rem, 8), dtype=np.int32)[None]
  if rem:
    iotar = iotar[:, :rem]
  iota2 = jnp.asarray(iota2)
  iotar = jnp.asarray(iotar)

  prop1 = _make_propagate(n_acc, C, n_g, two_partials=False)
  p = prop1(y, ep, zeros, iota2, iotar)
  prop2 = _make_propagate(n_acc, C, n_g, two_partials=True)
  p2 = prop2(p, ep, zeros, iota2, iotar)
  out = _make_final_combine(n_acc, C)(p2, b)
  return out[:N]


# speedup vs baseline: 14.5766x; 1.0758x over previous
import functools

import numpy as np

import jax
import jax.numpy as jnp
from jax import lax
from jax.experimental import pallas as pl
from jax.experimental.pallas import tpu as pltpu
from jax.experimental.pallas import tpu_sc as plsc

NC = 2
NS = 16
NW = NC * NS
G = 128
NB = 4
L = 16


def _matmul(x, wt, m_out):
  _, D = x.shape
  C = wt.shape[1]
  BM = m_out // 8
  assert BM % 8 == 0

  def body(x_ref, w_ref, o_ref):
    o_ref[...] = jnp.dot(x_ref[...], w_ref[...],
                         preferred_element_type=jnp.float32)

  return pl.pallas_call(
      body,
      grid=(m_out // BM,),
      in_specs=[
          pl.BlockSpec((BM, D), lambda i: (i, 0)),
          pl.BlockSpec((D, C), lambda i: (0, 0)),
      ],
      out_specs=pl.BlockSpec((BM, C), lambda i: (i, 0)),
      out_shape=jax.ShapeDtypeStruct((m_out, C), jnp.float32),
  )(x, wt)


def _make_propagate(n_acc, c, n_g, two_partials):
  mesh = plsc.VectorSubcoreMesh(core_axis_name="c", subcore_axis_name="s")
  rows_per_tile = n_acc // NS
  n_full = rows_per_tile // G
  rem = rows_per_tile - n_full * G
  assert n_acc % (NS * 8) == 0 and rem % 8 == 0
  assert n_g % NB == 0 and n_full <= NB

  @functools.partial(
      pl.kernel,
      out_type=jax.ShapeDtypeStruct((NC, n_acc, c), jnp.float32),
      mesh=mesh,
      scratch_types=[
          pltpu.VMEM((n_g, G), jnp.int32),
          pltpu.VMEM((n_g, G), jnp.int32),
          pltpu.VMEM((n_full, G), jnp.int32),
          pltpu.VMEM((max(rem, 8),), jnp.int32),
          [pltpu.VMEM((G, c), jnp.float32)] * NB,
          pltpu.VMEM((64, c), jnp.float32),
          pltpu.VMEM_SHARED((n_acc, c), jnp.float32),
          pltpu.VMEM_SHARED((n_acc, c), jnp.float32),
          [pltpu.SemaphoreType.DMA] * NB,
          [pltpu.SemaphoreType.DMA] * NB,
      ],
      compiler_params=pltpu.CompilerParams(use_tc_tiling_on_sc=False),
  )
  def propagate(y_hbm, ep_hbm, iota2_hbm, iotar_hbm, out_hbm,
                src_v, dst_v, idx_v, idx_r, bufs, zbuf, y_sp, acc,
                gsems, ssems):
    cid = lax.axis_index("c")
    sid = lax.axis_index("s")
    wid = sid * NC + cid
    r0 = sid * rows_per_tile
    slab = pl.ds(r0, rows_per_tile)

    def zbody(r, carry):
      for l in range(c // L):
        zbuf[r, pl.ds(l * L, L)] = jnp.zeros((L,), jnp.float32)
      return carry

    lax.fori_loop(0, 64, zbody, 0)
    nz = rows_per_tile // 64
    zr = rows_per_tile - nz * 64
    for k in range(nz):
      pltpu.async_copy(zbuf, acc.at[pl.ds(r0 + 64 * k, 64)], ssems[3])
    if zr:
      pltpu.async_copy(zbuf.at[pl.ds(0, zr)],
                       acc.at[pl.ds(r0 + nz * 64, zr)], ssems[3])
    if two_partials:
      for k in range(n_full):
        pltpu.async_copy(y_hbm.at[1, pl.ds(r0 + k * G, G)], bufs[k],
                         gsems[k])
      pltpu.async_copy(iota2_hbm.at[sid], idx_v, ssems[0])
      pltpu.async_copy(iotar_hbm.at[sid], idx_r, ssems[1])
      pltpu.sync_copy(y_hbm.at[0, slab], y_sp.at[slab])
      pltpu.make_async_copy(iota2_hbm.at[sid], idx_v, ssems[0]).wait()
      pltpu.make_async_copy(iotar_hbm.at[sid], idx_r, ssems[1]).wait()
      for k in range(n_full):
        pltpu.make_async_copy(y_hbm.at[1, pl.ds(r0 + k * G, G)],
                              bufs[k], gsems[k]).wait()
        pltpu.sync_copy(bufs[k], y_sp.at[idx_v.at[k]], add=True)
      if rem:
        pltpu.sync_copy(y_hbm.at[1, pl.ds(r0 + n_full * G, rem)],
                        bufs[0].at[pl.ds(0, rem)])
        pltpu.sync_copy(bufs[0].at[pl.ds(0, rem)], y_sp.at[idx_r],
                        add=True)
    else:
      pltpu.sync_copy(y_hbm.at[slab], y_sp.at[slab])
    pltpu.sync_copy(ep_hbm.at[0, wid], src_v)
    pltpu.sync_copy(ep_hbm.at[1, wid], dst_v)
    for k in range(nz):
      pltpu.make_async_copy(zbuf, acc.at[pl.ds(r0 + 64 * k, 64)],
                            ssems[3]).wait()
    if zr:
      pltpu.make_async_copy(zbuf.at[pl.ds(0, zr)],
                            acc.at[pl.ds(r0 + nz * 64, zr)],
                            ssems[3]).wait()
    plsc.subcore_barrier()

    for ph in range(NB):
      pltpu.async_copy(y_sp.at[src_v.at[ph]], bufs[ph], gsems[ph])

    def body(i, carry):
      base = i * NB
      for ph in range(NB):
        g = base + ph
        pltpu.make_async_copy(y_sp.at[src_v.at[g]], bufs[ph],
                              gsems[ph]).wait()
        pltpu.async_copy(bufs[ph], acc.at[dst_v.at[g]], ssems[ph],
                         add=True)
      for ph in range(NB):
        g = base + ph
        pltpu.make_async_copy(bufs[ph], acc.at[dst_v.at[g]],
                              ssems[ph]).wait()

        @pl.when(g + NB < n_g)
        def _():
          pltpu.async_copy(y_sp.at[src_v.at[g + NB]], bufs[ph],
                           gsems[ph])

      return carry

    lax.fori_loop(0, n_g // NB, body, 0)
    plsc.subcore_barrier()
    pltpu.sync_copy(acc.at[slab], out_hbm.at[cid, slab])

  return propagate


def _make_final_combine(n_out, c):
  mesh = plsc.VectorSubcoreMesh(core_axis_name="c", subcore_axis_name="s")
  rpw = -(-n_out // NW)
  last = n_out - (NW - 1) * rpw
  assert c % L == 0 and last > 0

  @functools.partial(
      pl.kernel,
      out_type=jax.ShapeDtypeStruct((n_out, c), jnp.float32),
      mesh=mesh,
      scratch_types=[
          pltpu.VMEM((rpw, c), jnp.float32),
          pltpu.VMEM((rpw, c), jnp.float32),
          pltpu.VMEM((c,), jnp.float32),
          pltpu.SemaphoreType.DMA,
          pltpu.SemaphoreType.DMA,
      ],
      compiler_params=pltpu.CompilerParams(use_tc_tiling_on_sc=False),
  )
  def combine(p_hbm, b_hbm, out_hbm, a_buf, b_buf, bias_v, sem_a, sem_b):
    cid = lax.axis_index("c")
    sid = lax.axis_index("s")
    wid = sid * NC + cid
    r0 = wid * rpw

    def do(cnt):
      rows = pl.ds(r0, cnt)
      pltpu.async_copy(p_hbm.at[0, rows], a_buf.at[pl.ds(0, cnt)], sem_a)
      pltpu.async_copy(p_hbm.at[1, rows], b_buf.at[pl.ds(0, cnt)], sem_b)
      pltpu.sync_copy(b_hbm, bias_v)
      bias = [bias_v[pl.ds(l * L, L)] for l in range(c // L)]
      pltpu.make_async_copy(p_hbm.at[0, rows], a_buf.at[pl.ds(0, cnt)],
                            sem_a).wait()
      pltpu.make_async_copy(p_hbm.at[1, rows], b_buf.at[pl.ds(0, cnt)],
                            sem_b).wait()

      def body(r, carry):
        for l in range(c // L):
          cols = pl.ds(l * L, L)
          a_buf[r, cols] = a_buf[r, cols] + b_buf[r, cols] + bias[l]
        return carry

      lax.fori_loop(0, cnt, body, 0)
      pltpu.sync_copy(a_buf.at[pl.ds(0, cnt)], out_hbm.at[rows])

    if rpw == last:
      do(rpw)
    else:
      @pl.when(wid < NW - 1)
      def _():
        do(rpw)

      @pl.when(wid == NW - 1)
      def _():
        do(last)

  return combine


def kernel(feat, edge_index, W, b):
  N, D = feat.shape
  C = W.shape[0]
  E = edge_index.shape[1]

  n_g = -(-E // (NW * G))
  n_g = -(-n_g // NB) * NB
  e_pad = NW * n_g * G
  n_acc = -(-(N + 1) // (NS * 8)) * NS * 8
  n_acc = -(-n_acc // NW) * NW

  ep = jnp.pad(edge_index, ((0, 0), (0, e_pad - E)), constant_values=N)
  ep = ep.reshape(2, NW, n_g, G)

  y = _matmul(feat, W.T, n_acc)
  rows_per_tile = n_acc // NS
  n_full = rows_per_tile // G
  rem = rows_per_tile - n_full * G
  base = np.arange(NS, dtype=np.int32)[:, None] * rows_per_tile
  iota2 = base[:, :, None] + np.arange(n_full * G, dtype=np.int32
                                       ).reshape(1, n_full, G)
  iotar = base + n_full * G + np.arange(max(rem, 8), dtype=np.int32)[None]
  if rem:
    iotar = iotar[:, :rem]
  iota2 = jnp.asarray(iota2)
  iotar = jnp.asarray(iotar)

  prop1 = _make_propagate(n_acc, C, n_g, two_partials=False)
  p = prop1(y, ep, iota2, iotar)
  prop2 = _make_propagate(n_acc, C, n_g, two_partials=True)
  p2 = prop2(p, ep, iota2, iotar)
  return _make_final_combine(N, C)(p2, b)


# speedup vs baseline: 14.9811x; 1.0277x over previous
import functools

import numpy as np

import jax
import jax.numpy as jnp
from jax import lax
from jax.experimental import pallas as pl
from jax.experimental.pallas import tpu as pltpu
from jax.experimental.pallas import tpu_sc as plsc

NC = 2
NS = 16
NW = NC * NS
G = 128
NB = 4
L = 16


def _matmul_prep(x, w, e, m_out, n_g):
  N, D = x.shape
  C = w.shape[0]
  E = e.shape[1]
  GRID = 8
  BM = m_out // GRID
  e_pad = NW * n_g * G
  EPB = e_pad // GRID
  WPB = NW // GRID
  assert BM % 8 == 0 and e_pad % GRID == 0

  def body(x_ref, w_ref, e_ref, o_ref, ep_ref):
    o_ref[...] = jnp.dot(x_ref[...], w_ref[...].T,
                         preferred_element_type=jnp.float32)
    i = pl.program_id(0)
    col = i * EPB + lax.broadcasted_iota(jnp.int32, (2, EPB), 1)
    v = jnp.where(col < E, e_ref[...], N)
    ep_ref[...] = v.reshape(2, WPB, n_g, G)

  return pl.pallas_call(
      body,
      grid=(GRID,),
      in_specs=[
          pl.BlockSpec((BM, D), lambda i: (i, 0)),
          pl.BlockSpec((C, D), lambda i: (0, 0)),
          pl.BlockSpec((2, EPB), lambda i: (0, i)),
      ],
      out_specs=[
          pl.BlockSpec((BM, C), lambda i: (i, 0)),
          pl.BlockSpec((2, WPB, n_g, G), lambda i: (0, i, 0, 0)),
      ],
      out_shape=[
          jax.ShapeDtypeStruct((m_out, C), jnp.float32),
          jax.ShapeDtypeStruct((2, NW, n_g, G), jnp.int32),
      ],
  )(x, w, e)


def _make_propagate(n_acc, c, n_g, two_partials):
  mesh = plsc.VectorSubcoreMesh(core_axis_name="c", subcore_axis_name="s")
  rows_per_tile = n_acc // NS
  n_full = rows_per_tile // G
  rem = rows_per_tile - n_full * G
  assert n_acc % (NS * 8) == 0 and rem % 8 == 0
  assert n_g % NB == 0 and n_full <= NB

  @functools.partial(
      pl.kernel,
      out_type=jax.ShapeDtypeStruct((NC, n_acc, c), jnp.float32),
      mesh=mesh,
      scratch_types=[
          pltpu.VMEM((n_g, G), jnp.int32),
          pltpu.VMEM((n_g, G), jnp.int32),
          pltpu.VMEM((n_full, G), jnp.int32),
          pltpu.VMEM((max(rem, 8),), jnp.int32),
          [pltpu.VMEM((G, c), jnp.float32)] * NB,
          pltpu.VMEM((64, c), jnp.float32),
          pltpu.VMEM_SHARED((n_acc, c), jnp.float32),
          pltpu.VMEM_SHARED((n_acc, c), jnp.float32),
          [pltpu.SemaphoreType.DMA] * NB,
          [pltpu.SemaphoreType.DMA] * NB,
      ],
      compiler_params=pltpu.CompilerParams(use_tc_tiling_on_sc=False),
  )
  def propagate(y_hbm, ep_hbm, iota2_hbm, iotar_hbm, out_hbm,
                src_v, dst_v, idx_v, idx_r, bufs, zbuf, y_sp, acc,
                gsems, ssems):
    cid = lax.axis_index("c")
    sid = lax.axis_index("s")
    wid = sid * NC + cid
    r0 = sid * rows_per_tile
    slab = pl.ds(r0, rows_per_tile)

    def zbody(r, carry):
      for l in range(c // L):
        zbuf[r, pl.ds(l * L, L)] = jnp.zeros((L,), jnp.float32)
      return carry

    lax.fori_loop(0, 64, zbody, 0)
    nz = rows_per_tile // 64
    zr = rows_per_tile - nz * 64
    for k in range(nz):
      pltpu.async_copy(zbuf, acc.at[pl.ds(r0 + 64 * k, 64)], ssems[3])
    if zr:
      pltpu.async_copy(zbuf.at[pl.ds(0, zr)],
                       acc.at[pl.ds(r0 + nz * 64, zr)], ssems[3])
    if two_partials:
      for k in range(n_full):
        pltpu.async_copy(y_hbm.at[1, pl.ds(r0 + k * G, G)], bufs[k],
                         gsems[k])
      pltpu.async_copy(iota2_hbm.at[sid], idx_v, ssems[0])
      pltpu.async_copy(iotar_hbm.at[sid], idx_r, ssems[1])
      pltpu.sync_copy(y_hbm.at[0, slab], y_sp.at[slab])
      pltpu.make_async_copy(iota2_hbm.at[sid], idx_v, ssems[0]).wait()
      pltpu.make_async_copy(iotar_hbm.at[sid], idx_r, ssems[1]).wait()
      for k in range(n_full):
        pltpu.make_async_copy(y_hbm.at[1, pl.ds(r0 + k * G, G)],
                              bufs[k], gsems[k]).wait()
        pltpu.sync_copy(bufs[k], y_sp.at[idx_v.at[k]], add=True)
      if rem:
        pltpu.sync_copy(y_hbm.at[1, pl.ds(r0 + n_full * G, rem)],
                        bufs[0].at[pl.ds(0, rem)])
        pltpu.sync_copy(bufs[0].at[pl.ds(0, rem)], y_sp.at[idx_r],
                        add=True)
    else:
      pltpu.sync_copy(y_hbm.at[slab], y_sp.at[slab])
    pltpu.sync_copy(ep_hbm.at[0, wid], src_v)
    pltpu.sync_copy(ep_hbm.at[1, wid], dst_v)
    for k in range(nz):
      pltpu.make_async_copy(zbuf, acc.at[pl.ds(r0 + 64 * k, 64)],
                            ssems[3]).wait()
    if zr:
      pltpu.make_async_copy(zbuf.at[pl.ds(0, zr)],
                            acc.at[pl.ds(r0 + nz * 64, zr)],
                            ssems[3]).wait()
    plsc.subcore_barrier()

    for ph in range(NB):
      pltpu.async_copy(y_sp.at[src_v.at[ph]], bufs[ph], gsems[ph])

    def body(i, carry):
      base = i * NB
      for ph in range(NB):
        g = base + ph
        pltpu.make_async_copy(y_sp.at[src_v.at[g]], bufs[ph],
                              gsems[ph]).wait()
        pltpu.async_copy(bufs[ph], acc.at[dst_v.at[g]], ssems[ph],
                         add=True)
      for ph in range(NB):
        g = base + ph
        pltpu.make_async_copy(bufs[ph], acc.at[dst_v.at[g]],
                              ssems[ph]).wait()

        @pl.when(g + NB < n_g)
        def _():
          pltpu.async_copy(y_sp.at[src_v.at[g + NB]], bufs[ph],
                           gsems[ph])

      return carry

    lax.fori_loop(0, n_g // NB, body, 0)
    plsc.subcore_barrier()
    pltpu.sync_copy(acc.at[slab], out_hbm.at[cid, slab])

  return propagate


def _make_final_combine(n_out, c):
  mesh = plsc.VectorSubcoreMesh(core_axis_name="c", subcore_axis_name="s")
  rpw = -(-n_out // NW)
  last = n_out - (NW - 1) * rpw
  assert c % L == 0 and last > 0

  @functools.partial(
      pl.kernel,
      out_type=jax.ShapeDtypeStruct((n_out, c), jnp.float32),
      mesh=mesh,
      scratch_types=[
          pltpu.VMEM((rpw, c), jnp.float32),
          pltpu.VMEM((rpw, c), jnp.float32),
          pltpu.VMEM((c,), jnp.float32),
          pltpu.SemaphoreType.DMA,
          pltpu.SemaphoreType.DMA,
      ],
      compiler_params=pltpu.CompilerParams(use_tc_tiling_on_sc=False),
  )
  def combine(p_hbm, b_hbm, out_hbm, a_buf, b_buf, bias_v, sem_a, sem_b):
    cid = lax.axis_index("c")
    sid = lax.axis_index("s")
    wid = sid * NC + cid
    r0 = wid * rpw

    def do(cnt):
      rows = pl.ds(r0, cnt)
      pltpu.async_copy(p_hbm.at[0, rows], a_buf.at[pl.ds(0, cnt)], sem_a)
      pltpu.async_copy(p_hbm.at[1, rows], b_buf.at[pl.ds(0, cnt)], sem_b)
      pltpu.sync_copy(b_hbm, bias_v)
      bias = [bias_v[pl.ds(l * L, L)] for l in range(c // L)]
      pltpu.make_async_copy(p_hbm.at[0, rows], a_buf.at[pl.ds(0, cnt)],
                            sem_a).wait()
      pltpu.make_async_copy(p_hbm.at[1, rows], b_buf.at[pl.ds(0, cnt)],
                            sem_b).wait()

      def body(r, carry):
        for l in range(c // L):
          cols = pl.ds(l * L, L)
          a_buf[r, cols] = a_buf[r, cols] + b_buf[r, cols] + bias[l]
        return carry

      lax.fori_loop(0, cnt, body, 0)
      pltpu.sync_copy(a_buf.at[pl.ds(0, cnt)], out_hbm.at[rows])

    if rpw == last:
      do(rpw)
    else:
      @pl.when(wid < NW - 1)
      def _():
        do(rpw)

      @pl.when(wid == NW - 1)
      def _():
        do(last)

  return combine


def kernel(feat, edge_index, W, b):
  N, D = feat.shape
  C = W.shape[0]
  E = edge_index.shape[1]

  n_g = -(-E // (NW * G))
  n_g = -(-n_g // NB) * NB
  e_pad = NW * n_g * G
  n_acc = -(-(N + 1) // (NS * 8)) * NS * 8
  n_acc = -(-n_acc // NW) * NW

  y, ep = _matmul_prep(feat, W, edge_index, n_acc, n_g)
  rows_per_tile = n_acc // NS
  n_full = rows_per_tile // G
  rem = rows_per_tile - n_full * G
  base = np.arange(NS, dtype=np.int32)[:, None] * rows_per_tile
  iota2 = base[:, :, None] + np.arange(n_full * G, dtype=np.int32
                                       ).reshape(1, n_full, G)
  iotar = base + n_full * G + np.arange(max(rem, 8), dtype=np.int32)[None]
  if rem:
    iotar = iotar[:, :rem]
  iota2 = jnp.asarray(iota2)
  iotar = jnp.asarray(iotar)

  prop1 = _make_propagate(n_acc, C, n_g, two_partials=False)
  p = prop1(y, ep, iota2, iotar)
  prop2 = _make_propagate(n_acc, C, n_g, two_partials=True)
  p2 = prop2(p, ep, iota2, iotar)
  return _make_final_combine(N, C)(p2, b)
